# baseline (speedup 1.0000x reference)
import functools

import jax
import jax.numpy as jnp
from jax import lax
from jax.experimental import pallas as pl
from jax.experimental.pallas import tpu as pltpu
from jax.experimental.pallas import tpu_sc as plsc

N = 10000
D = 128
N_GRAPHS = 128
N_CLASSES = 10
E = 320000

NC = 2
NS = 16
NW = NC * NS
L = 16

N_PAD = 10240
TB = 1024
CH = 50
EC = 200
SLAB = 40
RPT = N_PAD // NS
DEG_SLAB = 2000

_mesh = plsc.VectorSubcoreMesh(core_axis_name="c", subcore_axis_name="s")


@functools.partial(
    pl.kernel,
    out_type=jax.ShapeDtypeStruct((NW, N_PAD), jnp.float32),
    mesh=_mesh,
    scratch_types=[
        pltpu.VMEM((DEG_SLAB,), jnp.int32),
        pltpu.VMEM((N_PAD,), jnp.float32),
    ],
    compiler_params=pltpu.CompilerParams(needs_layout_passes=False),
)
def _deg_kernel(dst_hbm, out_hbm, idx_v, hist_v):
    c = lax.axis_index("c")
    s = lax.axis_index("s")
    w = s * NC + c

    zeros = jnp.zeros((L,), jnp.float32)

    def _zero(i, carry):
        hist_v[pl.ds(i * L, L)] = zeros
        return carry

    lax.fori_loop(0, N_PAD // L, _zero, 0)

    ones = jnp.ones((L,), jnp.float32)

    def _slab(si, carry):
        pltpu.sync_copy(dst_hbm.at[w].at[si], idx_v)

        def _accum(i, c2):
            idx = idx_v[pl.ds(i * L, L)]
            plsc.addupdate_scatter(hist_v, [idx], ones)
            return c2

        lax.fori_loop(0, DEG_SLAB // L, _accum, 0)
        return carry

    lax.fori_loop(0, (EC * CH) // DEG_SLAB, _slab, 0)
    pltpu.sync_copy(hist_v, out_hbm.at[w])


def _mm_body(x_ref, w_ref, h_ref):
    h_ref[...] = jnp.dot(x_ref[...], w_ref[...],
                         preferred_element_type=jnp.float32)


def _mm(x_pad, W):
    return pl.pallas_call(
        _mm_body,
        grid=(N_PAD // TB,),
        in_specs=[
            pl.BlockSpec((TB, D), lambda j: (j, 0)),
            pl.BlockSpec((D, D), lambda j: (0, 0)),
        ],
        out_specs=pl.BlockSpec((TB, D), lambda j: (j, 0)),
        out_shape=jax.ShapeDtypeStruct((N_PAD, D), jnp.float32),
    )(x_pad, W)


def _scale_body(h_ref, hist_ref, hn_ref, dinv_ref):
    deg = jnp.sum(hist_ref[...], axis=0) + 1.0
    dinv = lax.rsqrt(deg)
    hn_ref[...] = h_ref[...] * dinv[:, None]
    dinv_ref[...] = dinv[:, None]


def _scale(h, hist):
    return pl.pallas_call(
        _scale_body,
        grid=(N_PAD // TB,),
        in_specs=[
            pl.BlockSpec((TB, D), lambda j: (j, 0)),
            pl.BlockSpec((NW, TB), lambda j: (0, j)),
        ],
        out_specs=[
            pl.BlockSpec((TB, D), lambda j: (j, 0)),
            pl.BlockSpec((TB, 1), lambda j: (j, 0)),
        ],
        out_shape=[
            jax.ShapeDtypeStruct((N_PAD, D), jnp.float32),
            jax.ShapeDtypeStruct((N_PAD, 1), jnp.float32),
        ],
    )(h, hist)


@functools.partial(
    pl.kernel,
    out_type=jax.ShapeDtypeStruct((NC, N_PAD, D), jnp.float32),
    mesh=_mesh,
    scratch_types=[
        pltpu.VMEM((SLAB, CH), jnp.int32),
        pltpu.VMEM((SLAB, CH), jnp.int32),
        pltpu.VMEM((CH, D), jnp.float32),
        pltpu.VMEM((CH, D), jnp.float32),
        pltpu.VMEM((CH, D), jnp.float32),
        pltpu.VMEM((CH, D), jnp.float32),
        pltpu.VMEM_SHARED((N_PAD, D), jnp.float32),
        pltpu.SemaphoreType.DMA,
        pltpu.SemaphoreType.DMA,
        pltpu.SemaphoreType.DMA,
        pltpu.SemaphoreType.DMA,
        pltpu.SemaphoreType.DMA,
        pltpu.SemaphoreType.DMA,
        pltpu.SemaphoreType.DMA,
        pltpu.SemaphoreType.DMA,
    ],
    compiler_params=pltpu.CompilerParams(needs_layout_passes=False),
)
def _agg_kernel(src_hbm, dst_hbm, hn_hbm, out_hbm,
                src_v, dst_v, r0, r1, r2, r3, acc_sh,
                gs0, gs1, gs2, gs3, ss0, ss1, ss2, ss3):
    rows = (r0, r1, r2, r3)
    gsem = (gs0, gs1, gs2, gs3)
    ssem = (ss0, ss1, ss2, ss3)
    c = lax.axis_index("c")
    s = lax.axis_index("s")
    w = s * NC + c
    base = s * RPT

    @pl.when(c == 0)
    def _():
        pltpu.sync_copy(hn_hbm.at[pl.ds(base, RPT)],
                        acc_sh.at[pl.ds(base, RPT)])

    @pl.when(c == 1)
    def _():
        zeros = jnp.zeros((L,), jnp.float32)

        def _zbuf(i, carry):
            r0[i // (D // L), pl.ds((i % (D // L)) * L, L)] = zeros
            return carry

        lax.fori_loop(0, CH * (D // L), _zbuf, 0)

        def _zacc(i, carry):
            pltpu.sync_copy(r0.at[pl.ds(0, 40)],
                            acc_sh.at[pl.ds(base + i * 40, 40)])
            return carry

        lax.fori_loop(0, RPT // 40, _zacc, 0)

    plsc.subcore_barrier()

    def _wait_scatter(bi):
        pltpu.make_async_copy(rows[bi], acc_sh.at[dst_v.at[0]],
                              ssem[bi]).wait()

    def _wait_gather(bi):
        pltpu.make_async_copy(hn_hbm.at[src_v.at[0]], rows[bi],
                              gsem[bi]).wait()

    def _slab(si, carry):
        pltpu.sync_copy(src_hbm.at[w].at[pl.ds(si * SLAB, SLAB)], src_v)
        pltpu.sync_copy(dst_hbm.at[w].at[pl.ds(si * SLAB, SLAB)], dst_v)
        pltpu.async_copy(hn_hbm.at[src_v.at[0]], rows[0], gsem[0])
        pltpu.async_copy(hn_hbm.at[src_v.at[1]], rows[1], gsem[1])

        def _grp(g, c2):
            for b in range(4):
                lk = 4 * g + b
                nb = (b + 2) % 4
                if b >= 2:
                    _wait_scatter(nb)
                else:
                    @pl.when(g >= 1)
                    def _():
                        _wait_scatter(nb)
                if b >= 2:
                    @pl.when(g < SLAB // 4 - 1)
                    def _():
                        pltpu.async_copy(hn_hbm.at[src_v.at[lk + 2]],
                                         rows[nb], gsem[nb])
                else:
                    pltpu.async_copy(hn_hbm.at[src_v.at[lk + 2]],
                                     rows[nb], gsem[nb])
                _wait_gather(b)
                pltpu.async_copy(rows[b], acc_sh.at[dst_v.at[lk]],
                                 ssem[b], add=True)
            return c2

        lax.fori_loop(0, SLAB // 4, _grp, 0)
        _wait_scatter(2)
        _wait_scatter(3)
        return carry

    lax.fori_loop(0, EC // SLAB, _slab, 0)
    plsc.subcore_barrier()
    pltpu.sync_copy(acc_sh.at[pl.ds(base, RPT)],
                    out_hbm.at[c].at[pl.ds(base, RPT)])


def _tc2_body(p0_ref, p1_ref, dinv_ref, batch_ref, b_ref,
              wc_ref, bc_ref, out_ref, acc_ref, cnt_ref):
    j = pl.program_id(0)

    @pl.when(j == 0)
    def _():
        acc_ref[...] = jnp.zeros_like(acc_ref)
        cnt_ref[...] = jnp.zeros_like(cnt_ref)

    dinv = dinv_ref[...]
    act = jnp.maximum(
        dinv * (p0_ref[...] + p1_ref[...]) + b_ref[...], 0.0)
    gid = batch_ref[...]
    onehot = (gid == lax.broadcasted_iota(jnp.int32, (1, N_GRAPHS), 1)
              ).astype(jnp.float32)
    acc_ref[...] += lax.dot_general(
        onehot, act, (((0,), (0,)), ((), ())),
        preferred_element_type=jnp.float32)
    cnt_ref[...] += lax.dot_general(
        onehot, jnp.ones_like(act), (((0,), (0,)), ((), ())),
        preferred_element_type=jnp.float32)

    @pl.when(j == pl.num_programs(0) - 1)
    def _():
        pooled = acc_ref[...] / jnp.maximum(cnt_ref[...], 1.0)
        out_ref[...] = (
            jnp.dot(pooled, wc_ref[...], preferred_element_type=jnp.float32)
            + bc_ref[...])


def _tc2(p0, p1, dinv, batch_pad, b, wc_pad, bc_pad):
    return pl.pallas_call(
        _tc2_body,
        grid=(N_PAD // TB,),
        in_specs=[
            pl.BlockSpec((TB, D), lambda j: (j, 0)),
            pl.BlockSpec((TB, D), lambda j: (j, 0)),
            pl.BlockSpec((TB, 1), lambda j: (j, 0)),
            pl.BlockSpec((TB, 1), lambda j: (j, 0)),
            pl.BlockSpec((1, D), lambda j: (0, 0)),
            pl.BlockSpec((D, D), lambda j: (0, 0)),
            pl.BlockSpec((1, D), lambda j: (0, 0)),
        ],
        out_specs=pl.BlockSpec((N_GRAPHS, D), lambda j: (0, 0)),
        out_shape=jax.ShapeDtypeStruct((N_GRAPHS, D), jnp.float32),
        scratch_shapes=[
            pltpu.VMEM((N_GRAPHS, D), jnp.float32),
            pltpu.VMEM((N_GRAPHS, D), jnp.float32),
        ],
    )(p0, p1, dinv, batch_pad, b, wc_pad, bc_pad)


def kernel(x, edge_index, batch, W, b, W_cls, b_cls):
    src_ch = edge_index[0].reshape(NW, EC, CH)
    dst_ch = edge_index[1].reshape(NW, EC, CH)
    dst_deg = edge_index[1].reshape(NW, (EC * CH) // DEG_SLAB, DEG_SLAB)

    x_pad = jnp.concatenate(
        [x, jnp.zeros((N_PAD - N, D), jnp.float32)])
    batch_pad = jnp.concatenate(
        [batch, jnp.full((N_PAD - N,), N_GRAPHS, jnp.int32)]
    ).reshape(N_PAD, 1)
    wc_pad = jnp.zeros((D, D), jnp.float32).at[:, :N_CLASSES].set(W_cls)
    bc_pad = jnp.zeros((1, D), jnp.float32).at[0, :N_CLASSES].set(b_cls)

    hist = _deg_kernel(dst_deg)
    h = _mm(x_pad, W)
    hn, dinv = _scale(h, hist)
    partials = _agg_kernel(src_ch, dst_ch, hn)
    logits_pad = _tc2(partials[0], partials[1], dinv, batch_pad,
                      b.reshape(1, D), wc_pad, bc_pad)
    return logits_pad[:, :N_CLASSES]


# speedup vs baseline: 1.0751x; 1.0751x over previous
import functools

import jax
import jax.numpy as jnp
from jax import lax
from jax.experimental import pallas as pl
from jax.experimental.pallas import tpu as pltpu
from jax.experimental.pallas import tpu_sc as plsc

N = 10000
D = 128
N_GRAPHS = 128
N_CLASSES = 10
E = 320000

NC = 2
NS = 16
NW = NC * NS
L = 16

N_PAD = 10240
TB = 1024
CH = 50
EC = 200
SLAB = 40
RPT = N_PAD // NS
DEG_SLAB = 2000

_mesh = plsc.VectorSubcoreMesh(core_axis_name="c", subcore_axis_name="s")


@functools.partial(
    pl.kernel,
    out_type=jax.ShapeDtypeStruct((NW, N_PAD), jnp.float32),
    mesh=_mesh,
    scratch_types=[
        pltpu.VMEM((DEG_SLAB,), jnp.int32),
        pltpu.VMEM((N_PAD,), jnp.float32),
    ],
    compiler_params=pltpu.CompilerParams(needs_layout_passes=False),
)
def _deg_kernel(dst_hbm, out_hbm, idx_v, hist_v):
    c = lax.axis_index("c")
    s = lax.axis_index("s")
    w = s * NC + c

    zeros = jnp.zeros((L,), jnp.float32)

    def _zero(i, carry):
        hist_v[pl.ds(i * L, L)] = zeros
        return carry

    lax.fori_loop(0, N_PAD // L, _zero, 0)

    ones = jnp.ones((L,), jnp.float32)

    def _slab(si, carry):
        pltpu.sync_copy(dst_hbm.at[w].at[si], idx_v)

        def _accum(i, c2):
            idx = idx_v[pl.ds(i * L, L)]
            plsc.addupdate_scatter(hist_v, [idx], ones)
            return c2

        lax.fori_loop(0, DEG_SLAB // L, _accum, 0)
        return carry

    lax.fori_loop(0, (EC * CH) // DEG_SLAB, _slab, 0)
    pltpu.sync_copy(hist_v, out_hbm.at[w])


def _tc1_body(x_ref, w_ref, hist_ref, hn_ref, dinv_ref):
    deg = jnp.sum(hist_ref[...], axis=0) + 1.0
    dinv = lax.rsqrt(deg)
    h = jnp.dot(x_ref[...], w_ref[...], preferred_element_type=jnp.float32)
    hn_ref[...] = h * dinv[:, None]
    dinv_ref[...] = dinv[:, None]


def _tc1(x_pad, W, hist):
    return pl.pallas_call(
        _tc1_body,
        grid=(N_PAD // TB,),
        in_specs=[
            pl.BlockSpec((TB, D), lambda j: (j, 0)),
            pl.BlockSpec((D, D), lambda j: (0, 0)),
            pl.BlockSpec((NW, TB), lambda j: (0, j)),
        ],
        out_specs=[
            pl.BlockSpec((TB, D), lambda j: (j, 0)),
            pl.BlockSpec((TB, 1), lambda j: (j, 0)),
        ],
        out_shape=[
            jax.ShapeDtypeStruct((N_PAD, D), jnp.float32),
            jax.ShapeDtypeStruct((N_PAD, 1), jnp.float32),
        ],
    )(x_pad, W, hist)


@functools.partial(
    pl.kernel,
    out_type=jax.ShapeDtypeStruct((NC, N_PAD, D), jnp.float32),
    mesh=_mesh,
    scratch_types=[
        pltpu.VMEM((SLAB, CH), jnp.int32),
        pltpu.VMEM((SLAB, CH), jnp.int32),
        pltpu.VMEM((CH, D), jnp.float32),
        pltpu.VMEM((CH, D), jnp.float32),
        pltpu.VMEM((CH, D), jnp.float32),
        pltpu.VMEM((CH, D), jnp.float32),
        pltpu.VMEM((CH, D), jnp.float32),
        pltpu.VMEM_SHARED((N_PAD, D), jnp.float32),
        pltpu.SemaphoreType.DMA,
        pltpu.SemaphoreType.DMA,
        pltpu.SemaphoreType.DMA,
        pltpu.SemaphoreType.DMA,
        pltpu.SemaphoreType.DMA,
        pltpu.SemaphoreType.DMA,
        pltpu.SemaphoreType.DMA,
        pltpu.SemaphoreType.DMA,
        pltpu.SemaphoreType.DMA,
        pltpu.SemaphoreType.DMA,
    ],
    compiler_params=pltpu.CompilerParams(needs_layout_passes=False),
)
def _agg_kernel(src_hbm, dst_hbm, hn_hbm, out_hbm,
                src_v, dst_v, r0, r1, r2, r3, r4, acc_sh,
                gs0, gs1, gs2, gs3, gs4, ss0, ss1, ss2, ss3, ss4):
    rows = (r0, r1, r2, r3, r4)
    gsem = (gs0, gs1, gs2, gs3, gs4)
    ssem = (ss0, ss1, ss2, ss3, ss4)
    c = lax.axis_index("c")
    s = lax.axis_index("s")
    w = s * NC + c
    base = s * RPT

    @pl.when(c == 0)
    def _():
        pltpu.sync_copy(hn_hbm.at[pl.ds(base, RPT)],
                        acc_sh.at[pl.ds(base, RPT)])

    @pl.when(c == 1)
    def _():
        zeros = jnp.zeros((L,), jnp.float32)

        def _zbuf(i, carry):
            r0[i // (D // L), pl.ds((i % (D // L)) * L, L)] = zeros
            return carry

        lax.fori_loop(0, CH * (D // L), _zbuf, 0)

        def _zacc(i, carry):
            pltpu.sync_copy(r0.at[pl.ds(0, 40)],
                            acc_sh.at[pl.ds(base + i * 40, 40)])
            return carry

        lax.fori_loop(0, RPT // 40, _zacc, 0)

    plsc.subcore_barrier()

    def _wait_scatter(bi):
        pltpu.make_async_copy(rows[bi], acc_sh.at[dst_v.at[0]],
                              ssem[bi]).wait()

    def _wait_gather(bi):
        pltpu.make_async_copy(hn_hbm.at[src_v.at[0]], rows[bi],
                              gsem[bi]).wait()

    def _slab(si, carry):
        pltpu.sync_copy(src_hbm.at[w].at[pl.ds(si * SLAB, SLAB)], src_v)
        pltpu.sync_copy(dst_hbm.at[w].at[pl.ds(si * SLAB, SLAB)], dst_v)
        for b in range(3):
            pltpu.async_copy(hn_hbm.at[src_v.at[b]], rows[b], gsem[b])

        def _grp(g, c2):
            for b in range(5):
                lk = 5 * g + b
                nb = (b + 3) % 5
                if b >= 2:
                    _wait_scatter(nb)
                else:
                    @pl.when(g >= 1)
                    def _():
                        _wait_scatter(nb)
                if b >= 2:
                    @pl.when(g < SLAB // 5 - 1)
                    def _():
                        pltpu.async_copy(hn_hbm.at[src_v.at[lk + 3]],
                                         rows[nb], gsem[nb])
                else:
                    pltpu.async_copy(hn_hbm.at[src_v.at[lk + 3]],
                                     rows[nb], gsem[nb])
                _wait_gather(b)
                pltpu.async_copy(rows[b], acc_sh.at[dst_v.at[lk]],
                                 ssem[b], add=True)
            return c2

        lax.fori_loop(0, SLAB // 5, _grp, 0)
        _wait_scatter(3)
        _wait_scatter(4)
        return carry

    lax.fori_loop(0, EC // SLAB, _slab, 0)
    plsc.subcore_barrier()
    pltpu.sync_copy(acc_sh.at[pl.ds(base, RPT)],
                    out_hbm.at[c].at[pl.ds(base, RPT)])


def _tc2_body(p0_ref, p1_ref, dinv_ref, batch_ref, b_ref,
              wc_ref, bc_ref, out_ref, acc_ref, cnt_ref):
    j = pl.program_id(0)

    @pl.when(j == 0)
    def _():
        acc_ref[...] = jnp.zeros_like(acc_ref)
        cnt_ref[...] = jnp.zeros_like(cnt_ref)

    dinv = dinv_ref[...]
    act = jnp.maximum(
        dinv * (p0_ref[...] + p1_ref[...]) + b_ref[...], 0.0)
    gid = batch_ref[...]
    onehot = (gid == lax.broadcasted_iota(jnp.int32, (1, N_GRAPHS), 1)
              ).astype(jnp.float32)
    acc_ref[...] += lax.dot_general(
        onehot, act, (((0,), (0,)), ((), ())),
        preferred_element_type=jnp.float32)
    cnt_ref[...] += lax.dot_general(
        onehot, jnp.ones_like(act), (((0,), (0,)), ((), ())),
        preferred_element_type=jnp.float32)

    @pl.when(j == pl.num_programs(0) - 1)
    def _():
        pooled = acc_ref[...] / jnp.maximum(cnt_ref[...], 1.0)
        out_ref[...] = (
            jnp.dot(pooled, wc_ref[...], preferred_element_type=jnp.float32)
            + bc_ref[...])


def _tc2(p0, p1, dinv, batch_pad, b, wc_pad, bc_pad):
    return pl.pallas_call(
        _tc2_body,
        grid=(N_PAD // TB,),
        in_specs=[
            pl.BlockSpec((TB, D), lambda j: (j, 0)),
            pl.BlockSpec((TB, D), lambda j: (j, 0)),
            pl.BlockSpec((TB, 1), lambda j: (j, 0)),
            pl.BlockSpec((TB, 1), lambda j: (j, 0)),
            pl.BlockSpec((1, D), lambda j: (0, 0)),
            pl.BlockSpec((D, D), lambda j: (0, 0)),
            pl.BlockSpec((1, D), lambda j: (0, 0)),
        ],
        out_specs=pl.BlockSpec((N_GRAPHS, D), lambda j: (0, 0)),
        out_shape=jax.ShapeDtypeStruct((N_GRAPHS, D), jnp.float32),
        scratch_shapes=[
            pltpu.VMEM((N_GRAPHS, D), jnp.float32),
            pltpu.VMEM((N_GRAPHS, D), jnp.float32),
        ],
    )(p0, p1, dinv, batch_pad, b, wc_pad, bc_pad)


def kernel(x, edge_index, batch, W, b, W_cls, b_cls):
    src_ch = edge_index[0].reshape(NW, EC, CH)
    dst_ch = edge_index[1].reshape(NW, EC, CH)
    dst_deg = edge_index[1].reshape(NW, (EC * CH) // DEG_SLAB, DEG_SLAB)

    x_pad = jnp.concatenate(
        [x, jnp.zeros((N_PAD - N, D), jnp.float32)])
    batch_pad = jnp.concatenate(
        [batch, jnp.full((N_PAD - N,), N_GRAPHS, jnp.int32)]
    ).reshape(N_PAD, 1)
    wc_pad = jnp.zeros((D, D), jnp.float32).at[:, :N_CLASSES].set(W_cls)
    bc_pad = jnp.zeros((1, D), jnp.float32).at[0, :N_CLASSES].set(b_cls)

    hist = _deg_kernel(dst_deg)
    hn, dinv = _tc1(x_pad, W, hist)
    partials = _agg_kernel(src_ch, dst_ch, hn)
    logits_pad = _tc2(partials[0], partials[1], dinv, batch_pad,
                      b.reshape(1, D), wc_pad, bc_pad)
    return logits_pad[:, :N_CLASSES]


# speedup vs baseline: 1.1009x; 1.0240x over previous
import functools

import jax
import jax.numpy as jnp
from jax import lax
from jax.experimental import pallas as pl
from jax.experimental.pallas import tpu as pltpu
from jax.experimental.pallas import tpu_sc as plsc

N = 10000
D = 128
N_GRAPHS = 128
N_CLASSES = 10
E = 320000

NC = 2
NS = 16
NW = NC * NS
L = 16

N_PAD = 10240
TB = 1024
CH = 50
EC = 200
SLAB = 40
RPT = N_PAD // NS
DEG_SLAB = 2000

_mesh = plsc.VectorSubcoreMesh(core_axis_name="c", subcore_axis_name="s")


@functools.partial(
    pl.kernel,
    out_type=jax.ShapeDtypeStruct((NW, N_PAD), jnp.float32),
    mesh=_mesh,
    scratch_types=[
        pltpu.VMEM((DEG_SLAB,), jnp.int32),
        pltpu.VMEM((N_PAD,), jnp.float32),
    ],
    compiler_params=pltpu.CompilerParams(needs_layout_passes=False),
)
def _deg_kernel(dst_hbm, out_hbm, idx_v, hist_v):
    c = lax.axis_index("c")
    s = lax.axis_index("s")
    w = s * NC + c

    zeros = jnp.zeros((L,), jnp.float32)

    def _zero(i, carry):
        hist_v[pl.ds(i * L, L)] = zeros
        return carry

    lax.fori_loop(0, N_PAD // L, _zero, 0)

    ones = jnp.ones((L,), jnp.float32)

    def _slab(si, carry):
        pltpu.sync_copy(dst_hbm.at[w].at[si], idx_v)

        def _accum(i, c2):
            idx = idx_v[pl.ds(i * L, L)]
            plsc.addupdate_scatter(hist_v, [idx], ones)
            return c2

        lax.fori_loop(0, DEG_SLAB // L, _accum, 0)
        return carry

    lax.fori_loop(0, (EC * CH) // DEG_SLAB, _slab, 0)
    pltpu.sync_copy(hist_v, out_hbm.at[w])


def _tc1_body(x_ref, w_ref, hist_ref, hn_ref, dinv_ref):
    deg = jnp.sum(hist_ref[...], axis=0) + 1.0
    dinv = lax.rsqrt(deg)
    h = jnp.dot(x_ref[...], w_ref[...], preferred_element_type=jnp.float32)
    hn_ref[...] = h * dinv[:, None]
    dinv_ref[...] = dinv[:, None]


def _tc1(x_pad, W, hist):
    return pl.pallas_call(
        _tc1_body,
        grid=(N_PAD // TB,),
        in_specs=[
            pl.BlockSpec((TB, D), lambda j: (j, 0)),
            pl.BlockSpec((D, D), lambda j: (0, 0)),
            pl.BlockSpec((NW, TB), lambda j: (0, j)),
        ],
        out_specs=[
            pl.BlockSpec((TB, D), lambda j: (j, 0)),
            pl.BlockSpec((TB, 1), lambda j: (j, 0)),
        ],
        out_shape=[
            jax.ShapeDtypeStruct((N_PAD, D), jnp.float32),
            jax.ShapeDtypeStruct((N_PAD, 1), jnp.float32),
        ],
    )(x_pad, W, hist)


@functools.partial(
    pl.kernel,
    out_type=jax.ShapeDtypeStruct((NC, N_PAD, D), jnp.float32),
    mesh=_mesh,
    scratch_types=[
        pltpu.VMEM((SLAB, CH), jnp.int32),
        pltpu.VMEM((SLAB, CH), jnp.int32),
        pltpu.VMEM((CH, D), jnp.float32),
        pltpu.VMEM((CH, D), jnp.float32),
        pltpu.VMEM((CH, D), jnp.float32),
        pltpu.VMEM((CH, D), jnp.float32),
        pltpu.VMEM((CH, D), jnp.float32),
        pltpu.VMEM_SHARED((N_PAD, D), jnp.float32),
        pltpu.SemaphoreType.DMA,
        pltpu.SemaphoreType.DMA,
        pltpu.SemaphoreType.DMA,
        pltpu.SemaphoreType.DMA,
        pltpu.SemaphoreType.DMA,
        pltpu.SemaphoreType.DMA,
        pltpu.SemaphoreType.DMA,
        pltpu.SemaphoreType.DMA,
        pltpu.SemaphoreType.DMA,
        pltpu.SemaphoreType.DMA,
    ],
    compiler_params=pltpu.CompilerParams(needs_layout_passes=False),
)
def _agg_kernel(src_hbm, dst_hbm, hn_hbm, out_hbm,
                src_v, dst_v, r0, r1, r2, r3, r4, acc_sh,
                gs0, gs1, gs2, gs3, gs4, ss0, ss1, ss2, ss3, ss4):
    rows = (r0, r1, r2, r3, r4)
    gsem = (gs0, gs1, gs2, gs3, gs4)
    ssem = (ss0, ss1, ss2, ss3, ss4)
    c = lax.axis_index("c")
    s = lax.axis_index("s")
    w = s * NC + c
    base = s * RPT

    @pl.when(c == 0)
    def _():
        pltpu.sync_copy(hn_hbm.at[pl.ds(base, RPT)],
                        acc_sh.at[pl.ds(base, RPT)])

    @pl.when(c == 1)
    def _():
        zeros = jnp.zeros((L,), jnp.float32)

        def _zbuf(i, carry):
            r0[i // (D // L), pl.ds((i % (D // L)) * L, L)] = zeros
            return carry

        lax.fori_loop(0, CH * (D // L), _zbuf, 0)

        def _zacc(i, carry):
            pltpu.sync_copy(r0.at[pl.ds(0, 40)],
                            acc_sh.at[pl.ds(base + i * 40, 40)])
            return carry

        lax.fori_loop(0, RPT // 40, _zacc, 0)

    plsc.subcore_barrier()

    def _wait_scatter(bi):
        pltpu.make_async_copy(rows[bi], acc_sh.at[dst_v.at[0]],
                              ssem[bi]).wait()

    def _wait_gather(bi):
        pltpu.make_async_copy(hn_hbm.at[src_v.at[0]], rows[bi],
                              gsem[bi]).wait()

    def _slab(si, carry):
        pltpu.sync_copy(src_hbm.at[w].at[pl.ds(si * SLAB, SLAB)], src_v)
        pltpu.sync_copy(dst_hbm.at[w].at[pl.ds(si * SLAB, SLAB)], dst_v)
        for b in range(3):
            pltpu.async_copy(hn_hbm.at[src_v.at[b]], rows[b], gsem[b])

        def _grp(g, c2):
            for b in range(5):
                lk = 5 * g + b
                nb = (b + 3) % 5
                if b >= 2:
                    _wait_scatter(nb)
                else:
                    @pl.when(g >= 1)
                    def _():
                        _wait_scatter(nb)
                if b >= 2:
                    @pl.when(g < SLAB // 5 - 1)
                    def _():
                        pltpu.async_copy(hn_hbm.at[src_v.at[lk + 3]],
                                         rows[nb], gsem[nb])
                else:
                    pltpu.async_copy(hn_hbm.at[src_v.at[lk + 3]],
                                     rows[nb], gsem[nb])
                _wait_gather(b)
                pltpu.async_copy(rows[b], acc_sh.at[dst_v.at[lk]],
                                 ssem[b], add=True)
            return c2

        lax.fori_loop(0, SLAB // 5, _grp, 0)
        _wait_scatter(3)
        _wait_scatter(4)
        return carry

    lax.fori_loop(0, EC // SLAB, _slab, 0)
    plsc.subcore_barrier()
    pltpu.sync_copy(acc_sh.at[pl.ds(base, RPT)],
                    out_hbm.at[c].at[pl.ds(base, RPT)])


def _tc2_body(p0_ref, p1_ref, dinv_ref, batch_ref, b_ref,
              wc_ref, bc_ref, out_ref, acc_ref, cnt_ref):
    j = pl.program_id(0)

    @pl.when(j == 0)
    def _():
        acc_ref[...] = jnp.zeros_like(acc_ref)
        cnt_ref[...] = jnp.zeros_like(cnt_ref)

    dinv = dinv_ref[...]
    act = jnp.maximum(
        dinv * (p0_ref[...] + p1_ref[...]) + b_ref[...], 0.0)
    gid = batch_ref[...]
    valid = (lax.broadcasted_iota(jnp.int32, (TB, 1), 0) + j * TB) < N
    act = jnp.where(valid, act, 0.0)
    onehot = jnp.where(
        valid,
        (gid == lax.broadcasted_iota(jnp.int32, (1, N_GRAPHS), 1)
         ).astype(jnp.float32),
        0.0)
    acc_ref[...] += lax.dot_general(
        onehot, act, (((0,), (0,)), ((), ())),
        preferred_element_type=jnp.float32)
    cnt_ref[...] += lax.dot_general(
        onehot, jnp.ones_like(act), (((0,), (0,)), ((), ())),
        preferred_element_type=jnp.float32)

    @pl.when(j == pl.num_programs(0) - 1)
    def _():
        pooled = acc_ref[...] / jnp.maximum(cnt_ref[...], 1.0)
        out_ref[...] = (
            jnp.dot(pooled, wc_ref[...], preferred_element_type=jnp.float32)
            + bc_ref[...])


def _tc2(p0, p1, dinv, batch_pad, b, wc_pad, bc_pad):
    return pl.pallas_call(
        _tc2_body,
        grid=(N_PAD // TB,),
        in_specs=[
            pl.BlockSpec((TB, D), lambda j: (j, 0)),
            pl.BlockSpec((TB, D), lambda j: (j, 0)),
            pl.BlockSpec((TB, 1), lambda j: (j, 0)),
            pl.BlockSpec((TB, 1), lambda j: (j, 0)),
            pl.BlockSpec((1, D), lambda j: (0, 0)),
            pl.BlockSpec((D, D), lambda j: (0, 0)),
            pl.BlockSpec((1, D), lambda j: (0, 0)),
        ],
        out_specs=pl.BlockSpec((N_GRAPHS, D), lambda j: (0, 0)),
        out_shape=jax.ShapeDtypeStruct((N_GRAPHS, D), jnp.float32),
        scratch_shapes=[
            pltpu.VMEM((N_GRAPHS, D), jnp.float32),
            pltpu.VMEM((N_GRAPHS, D), jnp.float32),
        ],
    )(p0, p1, dinv, batch_pad, b, wc_pad, bc_pad)


def kernel(x, edge_index, batch, W, b, W_cls, b_cls):
    src_ch = edge_index[0].reshape(NW, EC, CH)
    dst_ch = edge_index[1].reshape(NW, EC, CH)
    dst_deg = edge_index[1].reshape(NW, (EC * CH) // DEG_SLAB, DEG_SLAB)

    batch2d = batch.reshape(N, 1)
    wc_pad = jnp.zeros((D, D), jnp.float32).at[:, :N_CLASSES].set(W_cls)
    bc_pad = jnp.zeros((1, D), jnp.float32).at[0, :N_CLASSES].set(b_cls)

    hist = _deg_kernel(dst_deg)
    hn, dinv = _tc1(x, W, hist)
    partials = _agg_kernel(src_ch, dst_ch, hn)
    logits_pad = _tc2(partials[0], partials[1], dinv, batch2d,
                      b.reshape(1, D), wc_pad, bc_pad)
    return logits_pad[:, :N_CLASSES]


# speedup vs baseline: 1.1029x; 1.0019x over previous
import functools

import jax
import jax.numpy as jnp
from jax import lax
from jax.experimental import pallas as pl
from jax.experimental.pallas import tpu as pltpu
from jax.experimental.pallas import tpu_sc as plsc

N = 10000
D = 128
N_GRAPHS = 128
N_CLASSES = 10
E = 320000

NC = 2
NS = 16
NW = NC * NS
L = 16

N_PAD = 10240
TB = 1024
CH = 50
EC = 200
SLAB = 40
RPT = N_PAD // NS
DEG_SLAB = 2000

_mesh = plsc.VectorSubcoreMesh(core_axis_name="c", subcore_axis_name="s")


@functools.partial(
    pl.kernel,
    out_type=jax.ShapeDtypeStruct((NW, N_PAD), jnp.float32),
    mesh=_mesh,
    scratch_types=[
        pltpu.VMEM((DEG_SLAB,), jnp.int32),
        pltpu.VMEM((N_PAD,), jnp.float32),
    ],
    compiler_params=pltpu.CompilerParams(needs_layout_passes=False),
)
def _deg_kernel(dst_hbm, out_hbm, idx_v, hist_v):
    c = lax.axis_index("c")
    s = lax.axis_index("s")
    w = s * NC + c

    zeros = jnp.zeros((L,), jnp.float32)

    def _zero(i, carry):
        for k in range(8):
            hist_v[pl.ds(i * 8 * L + k * L, L)] = zeros
        return carry

    lax.fori_loop(0, N_PAD // (8 * L), _zero, 0)

    ones = jnp.ones((L,), jnp.float32)

    def _slab(si, carry):
        pltpu.sync_copy(dst_hbm.at[w].at[si], idx_v)

        def _accum(i, c2):
            for k in range(5):
                idx = idx_v[pl.ds(i * 5 * L + k * L, L)]
                plsc.addupdate_scatter(hist_v, [idx], ones)
            return c2

        lax.fori_loop(0, DEG_SLAB // (5 * L), _accum, 0)
        return carry

    lax.fori_loop(0, (EC * CH) // DEG_SLAB, _slab, 0)
    pltpu.sync_copy(hist_v, out_hbm.at[w])


def _tc1_body(x_ref, w_ref, hist_ref, hn_ref, dinv_ref):
    deg = jnp.sum(hist_ref[...], axis=0) + 1.0
    dinv = lax.rsqrt(deg)
    h = jnp.dot(x_ref[...], w_ref[...], preferred_element_type=jnp.float32)
    hn_ref[...] = h * dinv[:, None]
    dinv_ref[...] = dinv[:, None]


def _tc1(x_pad, W, hist):
    return pl.pallas_call(
        _tc1_body,
        grid=(N_PAD // TB,),
        in_specs=[
            pl.BlockSpec((TB, D), lambda j: (j, 0)),
            pl.BlockSpec((D, D), lambda j: (0, 0)),
            pl.BlockSpec((NW, TB), lambda j: (0, j)),
        ],
        out_specs=[
            pl.BlockSpec((TB, D), lambda j: (j, 0)),
            pl.BlockSpec((TB, 1), lambda j: (j, 0)),
        ],
        out_shape=[
            jax.ShapeDtypeStruct((N_PAD, D), jnp.float32),
            jax.ShapeDtypeStruct((N_PAD, 1), jnp.float32),
        ],
    )(x_pad, W, hist)


@functools.partial(
    pl.kernel,
    out_type=jax.ShapeDtypeStruct((NC, N_PAD, D), jnp.float32),
    mesh=_mesh,
    scratch_types=[
        pltpu.VMEM((SLAB, CH), jnp.int32),
        pltpu.VMEM((SLAB, CH), jnp.int32),
        pltpu.VMEM((CH, D), jnp.float32),
        pltpu.VMEM((CH, D), jnp.float32),
        pltpu.VMEM((CH, D), jnp.float32),
        pltpu.VMEM((CH, D), jnp.float32),
        pltpu.VMEM((CH, D), jnp.float32),
        pltpu.VMEM_SHARED((N_PAD, D), jnp.float32),
        pltpu.SemaphoreType.DMA,
        pltpu.SemaphoreType.DMA,
        pltpu.SemaphoreType.DMA,
        pltpu.SemaphoreType.DMA,
        pltpu.SemaphoreType.DMA,
        pltpu.SemaphoreType.DMA,
        pltpu.SemaphoreType.DMA,
        pltpu.SemaphoreType.DMA,
        pltpu.SemaphoreType.DMA,
        pltpu.SemaphoreType.DMA,
    ],
    compiler_params=pltpu.CompilerParams(needs_layout_passes=False),
)
def _agg_kernel(src_hbm, dst_hbm, hn_hbm, out_hbm,
                src_v, dst_v, r0, r1, r2, r3, r4, acc_sh,
                gs0, gs1, gs2, gs3, gs4, ss0, ss1, ss2, ss3, ss4):
    rows = (r0, r1, r2, r3, r4)
    gsem = (gs0, gs1, gs2, gs3, gs4)
    ssem = (ss0, ss1, ss2, ss3, ss4)
    c = lax.axis_index("c")
    s = lax.axis_index("s")
    w = s * NC + c
    base = s * RPT

    @pl.when(c == 0)
    def _():
        pltpu.sync_copy(hn_hbm.at[pl.ds(base, RPT)],
                        acc_sh.at[pl.ds(base, RPT)])

    @pl.when(c == 1)
    def _():
        zeros = jnp.zeros((L,), jnp.float32)

        def _zbuf(i, carry):
            r0[i // (D // L), pl.ds((i % (D // L)) * L, L)] = zeros
            return carry

        lax.fori_loop(0, CH * (D // L), _zbuf, 0)

        def _zacc(i, carry):
            pltpu.sync_copy(r0.at[pl.ds(0, 40)],
                            acc_sh.at[pl.ds(base + i * 40, 40)])
            return carry

        lax.fori_loop(0, RPT // 40, _zacc, 0)

    plsc.subcore_barrier()

    def _wait_scatter(bi):
        pltpu.make_async_copy(rows[bi], acc_sh.at[dst_v.at[0]],
                              ssem[bi]).wait()

    def _wait_gather(bi):
        pltpu.make_async_copy(hn_hbm.at[src_v.at[0]], rows[bi],
                              gsem[bi]).wait()

    def _slab(si, carry):
        pltpu.sync_copy(src_hbm.at[w].at[pl.ds(si * SLAB, SLAB)], src_v)
        pltpu.sync_copy(dst_hbm.at[w].at[pl.ds(si * SLAB, SLAB)], dst_v)
        for b in range(3):
            pltpu.async_copy(hn_hbm.at[src_v.at[b]], rows[b], gsem[b])

        def _grp(g, c2):
            for b in range(5):
                lk = 5 * g + b
                nb = (b + 3) % 5
                if b >= 2:
                    _wait_scatter(nb)
                else:
                    @pl.when(g >= 1)
                    def _():
                        _wait_scatter(nb)
                if b >= 2:
                    @pl.when(g < SLAB // 5 - 1)
                    def _():
                        pltpu.async_copy(hn_hbm.at[src_v.at[lk + 3]],
                                         rows[nb], gsem[nb])
                else:
                    pltpu.async_copy(hn_hbm.at[src_v.at[lk + 3]],
                                     rows[nb], gsem[nb])
                _wait_gather(b)
                pltpu.async_copy(rows[b], acc_sh.at[dst_v.at[lk]],
                                 ssem[b], add=True)
            return c2

        lax.fori_loop(0, SLAB // 5, _grp, 0)
        _wait_scatter(3)
        _wait_scatter(4)
        return carry

    lax.fori_loop(0, EC // SLAB, _slab, 0)
    plsc.subcore_barrier()
    pltpu.sync_copy(acc_sh.at[pl.ds(base, RPT)],
                    out_hbm.at[c].at[pl.ds(base, RPT)])


def _tc2_body(p0_ref, p1_ref, dinv_ref, batch_ref, b_ref,
              wc_ref, bc_ref, out_ref, acc_ref, cnt_ref):
    j = pl.program_id(0)

    @pl.when(j == 0)
    def _():
        acc_ref[...] = jnp.zeros_like(acc_ref)
        cnt_ref[...] = jnp.zeros_like(cnt_ref)

    dinv = dinv_ref[...]
    act = jnp.maximum(
        dinv * (p0_ref[...] + p1_ref[...]) + b_ref[...], 0.0)
    gid = batch_ref[...]
    valid = (lax.broadcasted_iota(jnp.int32, (TB, 1), 0) + j * TB) < N
    act = jnp.where(valid, act, 0.0)
    onehot = jnp.where(
        valid,
        (gid == lax.broadcasted_iota(jnp.int32, (1, N_GRAPHS), 1)
         ).astype(jnp.float32),
        0.0)
    acc_ref[...] += lax.dot_general(
        onehot, act, (((0,), (0,)), ((), ())),
        preferred_element_type=jnp.float32)
    cnt_ref[...] += lax.dot_general(
        onehot, jnp.ones_like(act), (((0,), (0,)), ((), ())),
        preferred_element_type=jnp.float32)

    @pl.when(j == pl.num_programs(0) - 1)
    def _():
        pooled = acc_ref[...] / jnp.maximum(cnt_ref[...], 1.0)
        out_ref[...] = (
            jnp.dot(pooled, wc_ref[...], preferred_element_type=jnp.float32)
            + bc_ref[...])


def _tc2(p0, p1, dinv, batch_pad, b, wc_pad, bc_pad):
    return pl.pallas_call(
        _tc2_body,
        grid=(N_PAD // TB,),
        in_specs=[
            pl.BlockSpec((TB, D), lambda j: (j, 0)),
            pl.BlockSpec((TB, D), lambda j: (j, 0)),
            pl.BlockSpec((TB, 1), lambda j: (j, 0)),
            pl.BlockSpec((TB, 1), lambda j: (j, 0)),
            pl.BlockSpec((1, D), lambda j: (0, 0)),
            pl.BlockSpec((D, D), lambda j: (0, 0)),
            pl.BlockSpec((1, D), lambda j: (0, 0)),
        ],
        out_specs=pl.BlockSpec((N_GRAPHS, D), lambda j: (0, 0)),
        out_shape=jax.ShapeDtypeStruct((N_GRAPHS, D), jnp.float32),
        scratch_shapes=[
            pltpu.VMEM((N_GRAPHS, D), jnp.float32),
            pltpu.VMEM((N_GRAPHS, D), jnp.float32),
        ],
    )(p0, p1, dinv, batch_pad, b, wc_pad, bc_pad)


def kernel(x, edge_index, batch, W, b, W_cls, b_cls):
    src_ch = edge_index[0].reshape(NW, EC, CH)
    dst_ch = edge_index[1].reshape(NW, EC, CH)
    dst_deg = edge_index[1].reshape(NW, (EC * CH) // DEG_SLAB, DEG_SLAB)

    batch2d = batch.reshape(N, 1)
    wc_pad = jnp.zeros((D, D), jnp.float32).at[:, :N_CLASSES].set(W_cls)
    bc_pad = jnp.zeros((1, D), jnp.float32).at[0, :N_CLASSES].set(b_cls)

    hist = _deg_kernel(dst_deg)
    hn, dinv = _tc1(x, W, hist)
    partials = _agg_kernel(src_ch, dst_ch, hn)
    logits_pad = _tc2(partials[0], partials[1], dinv, batch2d,
                      b.reshape(1, D), wc_pad, bc_pad)
    return logits_pad[:, :N_CLASSES]
